# Initial kernel scaffold; baseline (speedup 1.0000x reference)
import jax
import jax.numpy as jnp
from jax.experimental import pallas as pl


def kernel(x, edge_index, W1, b1, W2, b2):
    raise NotImplementedError("write your pallas kernel here")


# speedup vs baseline: 21.3184x; 21.3184x over previous
import functools

import jax
import jax.numpy as jnp
from jax import lax
from jax.experimental import pallas as pl
from jax.experimental.pallas import tpu as pltpu
from jax.experimental.pallas import tpu_sc as plsc

N = 10000
D = 128
H = 16
A = 8
E = 320000

NTILES = 32
C = 128
CHUNKS = 79
EP = NTILES * CHUNKS * C
ROWS_PER_TILE = 632
NP = 16 * ROWS_PER_TILE

_mesh = plsc.VectorSubcoreMesh(core_axis_name="c", subcore_axis_name="s")
_sc_params = pltpu.CompilerParams(use_tc_tiling_on_sc=False)


@functools.partial(
    pl.kernel,
    mesh=_mesh,
    compiler_params=_sc_params,
    out_type=jax.ShapeDtypeStruct((2 * NP, H), jnp.float32),
    scratch_types=[
        pltpu.VMEM((C,), jnp.int32),
        pltpu.VMEM((C, H), jnp.float32),
        pltpu.VMEM_SHARED((NP, H), jnp.float32),
    ],
)
def _sc_degree(dst_hbm, ones_hbm, zeros_hbm, out_hbm, dst_v, ones_v, acc):
    c = lax.axis_index("c")
    s = lax.axis_index("s")
    wid = s * 2 + c
    r0 = s * ROWS_PER_TILE
    pltpu.sync_copy(zeros_hbm.at[pl.ds(r0, ROWS_PER_TILE)],
                    acc.at[pl.ds(r0, ROWS_PER_TILE)])
    pltpu.sync_copy(ones_hbm, ones_v)
    plsc.subcore_barrier()
    base = wid * (CHUNKS * C)

    def body(j, carry):
        off = base + j * C
        pltpu.sync_copy(dst_hbm.at[pl.ds(off, C)], dst_v)
        pltpu.sync_copy(ones_v, acc.at[dst_v], add=True)
        return carry

    lax.fori_loop(0, CHUNKS, body, 0)
    plsc.subcore_barrier()
    pltpu.sync_copy(acc.at[pl.ds(r0, ROWS_PER_TILE)],
                    out_hbm.at[pl.ds(c * NP + r0, ROWS_PER_TILE)])


@functools.partial(
    pl.kernel,
    mesh=_mesh,
    compiler_params=_sc_params,
    out_type=jax.ShapeDtypeStruct((2 * NP, H), jnp.float32),
    scratch_types=[
        pltpu.VMEM((C,), jnp.int32),
        pltpu.VMEM((C,), jnp.int32),
        pltpu.VMEM((C, H), jnp.float32),
        pltpu.VMEM_SHARED((NP, H), jnp.float32),
        pltpu.SemaphoreType.DMA,
    ],
)
def _sc_aggregate(src_hbm, dst_hbm, g_hbm, zeros_hbm, out_hbm,
                  src_v, dst_v, rows_v, acc, sem):
    c = lax.axis_index("c")
    s = lax.axis_index("s")
    wid = s * 2 + c
    r0 = s * ROWS_PER_TILE
    pltpu.sync_copy(zeros_hbm.at[pl.ds(r0, ROWS_PER_TILE)],
                    acc.at[pl.ds(r0, ROWS_PER_TILE)])
    plsc.subcore_barrier()
    base = wid * (CHUNKS * C)

    def body(j, carry):
        off = base + j * C
        pltpu.sync_copy(src_hbm.at[pl.ds(off, C)], src_v)
        pltpu.sync_copy(dst_hbm.at[pl.ds(off, C)], dst_v)
        pltpu.async_copy(g_hbm.at[src_v], rows_v, sem).wait()
        pltpu.sync_copy(rows_v, acc.at[dst_v], add=True)
        return carry

    lax.fori_loop(0, CHUNKS, body, 0)
    plsc.subcore_barrier()
    pltpu.sync_copy(acc.at[pl.ds(r0, ROWS_PER_TILE)],
                    out_hbm.at[pl.ds(c * NP + r0, ROWS_PER_TILE)])


def _tc_first(x_ref, w1_ref, degp_ref, g1_ref, dinv_ref):
    deg = degp_ref[0:NP, :] + degp_ref[NP:2 * NP, :] + 1.0
    dinv = lax.rsqrt(deg)
    dinv_ref[...] = dinv
    h1 = jnp.dot(x_ref[...], w1_ref[...], preferred_element_type=jnp.float32)
    g1_ref[0:N, :] = h1 * dinv[0:N, :]
    g1_ref[N:NP, :] = jnp.zeros((NP - N, H), jnp.float32)


def _tc_mid(s1p_ref, g1_ref, dinv_ref, b1_ref, w2_ref, g2_ref):
    s = s1p_ref[0:NP, :] + s1p_ref[NP:2 * NP, :] + g1_ref[...]
    h = jnp.maximum(s * dinv_ref[...] + b1_ref[...], 0.0)
    h2 = jnp.dot(h, w2_ref[...], preferred_element_type=jnp.float32)
    g2_ref[...] = h2 * dinv_ref[...]


def _tc_last(s2p_ref, g2_ref, dinv_ref, b2_ref, out_ref):
    s = s2p_ref[0:NP, :] + s2p_ref[NP:2 * NP, :] + g2_ref[...]
    out_ref[...] = s * dinv_ref[...] + b2_ref[...]


_tc_first_call = pl.pallas_call(
    _tc_first,
    out_shape=(jax.ShapeDtypeStruct((NP, H), jnp.float32),
               jax.ShapeDtypeStruct((NP, H), jnp.float32)),
)

_tc_mid_call = pl.pallas_call(
    _tc_mid,
    out_shape=jax.ShapeDtypeStruct((NP, H), jnp.float32),
)

_tc_last_call = pl.pallas_call(
    _tc_last,
    out_shape=jax.ShapeDtypeStruct((NP, H), jnp.float32),
)


def kernel(x, edge_index, W1, b1, W2, b2):
    pad = EP - E
    src = jnp.concatenate([edge_index[0],
                           jnp.full((pad,), N, jnp.int32)])
    dst = jnp.concatenate([edge_index[1],
                           jnp.full((pad,), N, jnp.int32)])
    zeros = jnp.zeros((NP, H), jnp.float32)
    ones = jnp.ones((C, H), jnp.float32)
    W2p = jnp.pad(W2, ((0, 0), (0, H - A)))
    b2p = jnp.pad(b2, (0, H - A)).reshape(1, H)
    b1r = b1.reshape(1, H)

    degp = _sc_degree(dst, ones, zeros)
    g1, dinv = _tc_first_call(x, W1, degp)
    s1p = _sc_aggregate(src, dst, g1, zeros)
    g2 = _tc_mid_call(s1p, g1, dinv, b1r, W2p)
    s2p = _sc_aggregate(src, dst, g2, zeros)
    out = _tc_last_call(s2p, g2, dinv, b2p)
    return out[0:N, 0:A]


# speedup vs baseline: 31.5836x; 1.4815x over previous
import functools

import jax
import jax.numpy as jnp
from jax import lax
from jax.experimental import pallas as pl
from jax.experimental.pallas import tpu as pltpu
from jax.experimental.pallas import tpu_sc as plsc

N = 10000
D = 128
H = 16
A = 8
E = 320000

NTILES = 32
C = 128
K = 8
NITER = 10
CHUNKS = K * NITER
EP = NTILES * CHUNKS * C
ECHUNKS = EP // C
ROWS_PER_TILE = 632
NP = 16 * ROWS_PER_TILE

_mesh = plsc.VectorSubcoreMesh(core_axis_name="c", subcore_axis_name="s")
_sc_params = pltpu.CompilerParams(use_tc_tiling_on_sc=False)


@functools.partial(
    pl.kernel,
    mesh=_mesh,
    compiler_params=_sc_params,
    out_type=jax.ShapeDtypeStruct((2 * NP, H), jnp.float32),
    scratch_types=[
        pltpu.VMEM((K, C), jnp.int32),
        pltpu.VMEM((C, H), jnp.float32),
        pltpu.VMEM_SHARED((NP, H), jnp.float32),
        pltpu.SemaphoreType.DMA,
        pltpu.SemaphoreType.DMA,
    ],
)
def _sc_degree(dst_hbm, ones_hbm, zeros_hbm, out_hbm,
               dst_v, ones_v, acc, isem, ssem):
    c = lax.axis_index("c")
    s = lax.axis_index("s")
    wid = s * 2 + c
    r0 = s * ROWS_PER_TILE
    pltpu.sync_copy(zeros_hbm.at[pl.ds(r0, ROWS_PER_TILE)],
                    acc.at[pl.ds(r0, ROWS_PER_TILE)])
    pltpu.sync_copy(ones_hbm, ones_v)
    plsc.subcore_barrier()
    base = wid * CHUNKS

    def body(i, carry):
        row0 = base + i * K
        pltpu.async_copy(dst_hbm.at[pl.ds(row0, K)], dst_v, isem).wait()
        scat = [pltpu.async_copy(ones_v, acc.at[dst_v.at[j]], ssem, add=True)
                for j in range(K)]
        for d in scat:
            d.wait()
        return carry

    lax.fori_loop(0, NITER, body, 0)
    plsc.subcore_barrier()
    pltpu.sync_copy(acc.at[pl.ds(r0, ROWS_PER_TILE)],
                    out_hbm.at[pl.ds(c * NP + r0, ROWS_PER_TILE)])


@functools.partial(
    pl.kernel,
    mesh=_mesh,
    compiler_params=_sc_params,
    out_type=jax.ShapeDtypeStruct((2 * NP, H), jnp.float32),
    scratch_types=[
        pltpu.VMEM((K, C), jnp.int32),
        pltpu.VMEM((K, C), jnp.int32),
        pltpu.VMEM((K, C, H), jnp.float32),
        pltpu.VMEM_SHARED((NP, H), jnp.float32),
        pltpu.SemaphoreType.DMA,
        pltpu.SemaphoreType.DMA,
        pltpu.SemaphoreType.DMA,
    ],
)
def _sc_aggregate(src_hbm, dst_hbm, g_hbm, zeros_hbm, out_hbm,
                  src_v, dst_v, rows_v, acc, isem, gsem, ssem):
    c = lax.axis_index("c")
    s = lax.axis_index("s")
    wid = s * 2 + c
    r0 = s * ROWS_PER_TILE
    pltpu.sync_copy(zeros_hbm.at[pl.ds(r0, ROWS_PER_TILE)],
                    acc.at[pl.ds(r0, ROWS_PER_TILE)])
    plsc.subcore_barrier()
    base = wid * CHUNKS

    def body(i, carry):
        row0 = base + i * K
        ic1 = pltpu.async_copy(src_hbm.at[pl.ds(row0, K)], src_v, isem)
        ic2 = pltpu.async_copy(dst_hbm.at[pl.ds(row0, K)], dst_v, isem)
        ic1.wait()
        ic2.wait()
        gat = [pltpu.async_copy(g_hbm.at[src_v.at[j]], rows_v.at[j], gsem)
               for j in range(K)]
        for d in gat:
            d.wait()
        scat = [pltpu.async_copy(rows_v.at[j], acc.at[dst_v.at[j]], ssem,
                                 add=True)
                for j in range(K)]
        for d in scat:
            d.wait()
        return carry

    lax.fori_loop(0, NITER, body, 0)
    plsc.subcore_barrier()
    pltpu.sync_copy(acc.at[pl.ds(r0, ROWS_PER_TILE)],
                    out_hbm.at[pl.ds(c * NP + r0, ROWS_PER_TILE)])


def _tc_first(x_ref, w1_ref, degp_ref, g1_ref, dinv_ref):
    deg = degp_ref[0:NP, :] + degp_ref[NP:2 * NP, :] + 1.0
    dinv = lax.rsqrt(deg)
    dinv_ref[...] = dinv
    h1 = jnp.dot(x_ref[...], w1_ref[...], preferred_element_type=jnp.float32)
    g1_ref[0:N, :] = h1 * dinv[0:N, :]
    g1_ref[N:NP, :] = jnp.zeros((NP - N, H), jnp.float32)


def _tc_mid(s1p_ref, g1_ref, dinv_ref, b1_ref, w2_ref, g2_ref):
    s = s1p_ref[0:NP, :] + s1p_ref[NP:2 * NP, :] + g1_ref[...]
    h = jnp.maximum(s * dinv_ref[...] + b1_ref[...], 0.0)
    h2 = jnp.dot(h, w2_ref[...], preferred_element_type=jnp.float32)
    g2_ref[...] = h2 * dinv_ref[...]


def _tc_last(s2p_ref, g2_ref, dinv_ref, b2_ref, out_ref):
    s = s2p_ref[0:NP, :] + s2p_ref[NP:2 * NP, :] + g2_ref[...]
    out_ref[...] = s * dinv_ref[...] + b2_ref[...]


_tc_first_call = pl.pallas_call(
    _tc_first,
    out_shape=(jax.ShapeDtypeStruct((NP, H), jnp.float32),
               jax.ShapeDtypeStruct((NP, H), jnp.float32)),
)

_tc_mid_call = pl.pallas_call(
    _tc_mid,
    out_shape=jax.ShapeDtypeStruct((NP, H), jnp.float32),
)

_tc_last_call = pl.pallas_call(
    _tc_last,
    out_shape=jax.ShapeDtypeStruct((NP, H), jnp.float32),
)


def kernel(x, edge_index, W1, b1, W2, b2):
    pad = EP - E
    src = jnp.concatenate([edge_index[0],
                           jnp.full((pad,), N, jnp.int32)]).reshape(ECHUNKS, C)
    dst = jnp.concatenate([edge_index[1],
                           jnp.full((pad,), N, jnp.int32)]).reshape(ECHUNKS, C)
    zeros = jnp.zeros((NP, H), jnp.float32)
    ones = jnp.ones((C, H), jnp.float32)
    W2p = jnp.pad(W2, ((0, 0), (0, H - A)))
    b2p = jnp.pad(b2, (0, H - A)).reshape(1, H)
    b1r = b1.reshape(1, H)

    degp = _sc_degree(dst, ones, zeros)
    g1, dinv = _tc_first_call(x, W1, degp)
    s1p = _sc_aggregate(src, dst, g1, zeros)
    g2 = _tc_mid_call(s1p, g1, dinv, b1r, W2p)
    s2p = _sc_aggregate(src, dst, g2, zeros)
    out = _tc_last_call(s2p, g2, dinv, b2p)
    return out[0:N, 0:A]


# speedup vs baseline: 53.5523x; 1.6956x over previous
import functools

import jax
import jax.numpy as jnp
from jax import lax
from jax.experimental import pallas as pl
from jax.experimental.pallas import tpu as pltpu
from jax.experimental.pallas import tpu_sc as plsc

N = 10000
D = 128
H = 16
A = 8
E = 320000

NTILES = 32
C = 128
K = 8
NPAIR = 5
NITER = 2 * NPAIR
CHUNKS = K * NITER
EP = NTILES * CHUNKS * C
ECHUNKS = EP // C + K
ROWS_PER_TILE = 632
NP = 16 * ROWS_PER_TILE

_mesh = plsc.VectorSubcoreMesh(core_axis_name="c", subcore_axis_name="s")
_sc_params = pltpu.CompilerParams(use_tc_tiling_on_sc=False)


@functools.partial(
    pl.kernel,
    mesh=_mesh,
    compiler_params=_sc_params,
    out_type=jax.ShapeDtypeStruct((2 * NP, H), jnp.float32),
    scratch_types=[
        pltpu.VMEM((K, C), jnp.int32),
        pltpu.VMEM((C, H), jnp.float32),
        pltpu.VMEM_SHARED((NP, H), jnp.float32),
        pltpu.SemaphoreType.DMA,
        pltpu.SemaphoreType.DMA,
    ],
)
def _sc_degree(dst_hbm, ones_hbm, zeros_hbm, out_hbm,
               dst_v, ones_v, acc, isem, ssem):
    c = lax.axis_index("c")
    s = lax.axis_index("s")
    wid = s * 2 + c
    r0 = s * ROWS_PER_TILE
    pltpu.sync_copy(zeros_hbm.at[pl.ds(r0, ROWS_PER_TILE)],
                    acc.at[pl.ds(r0, ROWS_PER_TILE)])
    pltpu.sync_copy(ones_hbm, ones_v)
    plsc.subcore_barrier()
    base = wid * CHUNKS

    def body(i, carry):
        row0 = base + i * K
        pltpu.async_copy(dst_hbm.at[pl.ds(row0, K)], dst_v, isem).wait()
        scat = [pltpu.async_copy(ones_v, acc.at[dst_v.at[j]], ssem, add=True)
                for j in range(K)]
        for d in scat:
            d.wait()
        return carry

    lax.fori_loop(0, NITER, body, 0)
    plsc.subcore_barrier()
    pltpu.sync_copy(acc.at[pl.ds(r0, ROWS_PER_TILE)],
                    out_hbm.at[pl.ds(c * NP + r0, ROWS_PER_TILE)])


@functools.partial(
    pl.kernel,
    mesh=_mesh,
    compiler_params=_sc_params,
    out_type=jax.ShapeDtypeStruct((2 * NP, H), jnp.float32),
    scratch_types=[
        pltpu.VMEM((2, K, C), jnp.int32),
        pltpu.VMEM((2, K, C), jnp.int32),
        pltpu.VMEM((2, K, C, H), jnp.float32),
        pltpu.VMEM_SHARED((NP, H), jnp.float32),
        pltpu.VMEM_SHARED((NP, H), jnp.float32),
        pltpu.SemaphoreType.DMA,
        pltpu.SemaphoreType.DMA,
        pltpu.SemaphoreType.DMA,
        pltpu.SemaphoreType.DMA,
    ],
)
def _sc_aggregate(src_hbm, dst_hbm, g_hbm, zeros_hbm, out_hbm,
                  src_v, dst_v, rows_v, gtab, acc, isem0, isem1, gsem, ssem):
    c = lax.axis_index("c")
    s = lax.axis_index("s")
    wid = s * 2 + c
    r0 = s * ROWS_PER_TILE
    pltpu.sync_copy(zeros_hbm.at[pl.ds(r0, ROWS_PER_TILE)],
                    acc.at[pl.ds(r0, ROWS_PER_TILE)])
    pltpu.sync_copy(g_hbm.at[pl.ds(r0, ROWS_PER_TILE)],
                    gtab.at[pl.ds(r0, ROWS_PER_TILE)])
    plsc.subcore_barrier()
    base = wid * CHUNKS

    def idx_copy(row0, b, sem):
        return (pltpu.async_copy(src_hbm.at[pl.ds(row0, K)], src_v.at[b],
                                 sem),
                pltpu.async_copy(dst_hbm.at[pl.ds(row0, K)], dst_v.at[b],
                                 sem))

    def fire_gathers(b):
        return [pltpu.async_copy(gtab.at[src_v.at[b, j]], rows_v.at[b, j],
                                 gsem)
                for j in range(K)]

    def fire_scatters(b):
        return [pltpu.async_copy(rows_v.at[b, j], acc.at[dst_v.at[b, j]],
                                 ssem, add=True)
                for j in range(K)]

    idx_copy(base, 0, isem0)

    def body(i, carry):
        row0 = base + (2 * i) * K
        i1 = idx_copy(row0 + K, 1, isem1)
        pltpu.make_async_copy(src_hbm.at[pl.ds(row0, K)], src_v.at[0],
                              isem0).wait()
        pltpu.make_async_copy(dst_hbm.at[pl.ds(row0, K)], dst_v.at[0],
                              isem0).wait()
        ga = fire_gathers(0)
        for d in ga:
            d.wait()
        sa = fire_scatters(0)
        i1[0].wait()
        i1[1].wait()
        gb = fire_gathers(1)
        for d in gb:
            d.wait()
        sb = fire_scatters(1)
        for d in sa:
            d.wait()
        idx_copy(row0 + 2 * K, 0, isem0)
        for d in sb:
            d.wait()
        return carry

    lax.fori_loop(0, NPAIR, body, 0)
    pltpu.make_async_copy(src_hbm.at[pl.ds(base + CHUNKS, K)], src_v.at[0],
                          isem0).wait()
    pltpu.make_async_copy(dst_hbm.at[pl.ds(base + CHUNKS, K)], dst_v.at[0],
                          isem0).wait()
    plsc.subcore_barrier()
    pltpu.sync_copy(acc.at[pl.ds(r0, ROWS_PER_TILE)],
                    out_hbm.at[pl.ds(c * NP + r0, ROWS_PER_TILE)])


def _tc_first(x_ref, w1_ref, degp_ref, g1_ref, dinv_ref):
    deg = degp_ref[0:NP, :] + degp_ref[NP:2 * NP, :] + 1.0
    dinv = lax.rsqrt(deg)
    dinv_ref[...] = dinv
    h1 = jnp.dot(x_ref[...], w1_ref[...], preferred_element_type=jnp.float32)
    g1_ref[0:N, :] = h1 * dinv[0:N, :]
    g1_ref[N:NP, :] = jnp.zeros((NP - N, H), jnp.float32)


def _tc_mid(s1p_ref, g1_ref, dinv_ref, b1_ref, w2_ref, g2_ref):
    s = s1p_ref[0:NP, :] + s1p_ref[NP:2 * NP, :] + g1_ref[...]
    h = jnp.maximum(s * dinv_ref[...] + b1_ref[...], 0.0)
    h2 = jnp.dot(h, w2_ref[...], preferred_element_type=jnp.float32)
    g2_ref[...] = h2 * dinv_ref[...]


def _tc_last(s2p_ref, g2_ref, dinv_ref, b2_ref, out_ref):
    s = s2p_ref[0:NP, :] + s2p_ref[NP:2 * NP, :] + g2_ref[...]
    out_ref[...] = s * dinv_ref[...] + b2_ref[...]


_tc_first_call = pl.pallas_call(
    _tc_first,
    out_shape=(jax.ShapeDtypeStruct((NP, H), jnp.float32),
               jax.ShapeDtypeStruct((NP, H), jnp.float32)),
)

_tc_mid_call = pl.pallas_call(
    _tc_mid,
    out_shape=jax.ShapeDtypeStruct((NP, H), jnp.float32),
)

_tc_last_call = pl.pallas_call(
    _tc_last,
    out_shape=jax.ShapeDtypeStruct((NP, H), jnp.float32),
)


def kernel(x, edge_index, W1, b1, W2, b2):
    pad = ECHUNKS * C - E
    src = jnp.concatenate([edge_index[0],
                           jnp.full((pad,), N, jnp.int32)]).reshape(ECHUNKS, C)
    dst = jnp.concatenate([edge_index[1],
                           jnp.full((pad,), N, jnp.int32)]).reshape(ECHUNKS, C)
    zeros = jnp.zeros((NP, H), jnp.float32)
    ones = jnp.ones((C, H), jnp.float32)
    W2p = jnp.pad(W2, ((0, 0), (0, H - A)))
    b2p = jnp.pad(b2, (0, H - A)).reshape(1, H)
    b1r = b1.reshape(1, H)

    degp = _sc_degree(dst, ones, zeros)
    g1, dinv = _tc_first_call(x, W1, degp)
    s1p = _sc_aggregate(src, dst, g1, zeros)
    g2 = _tc_mid_call(s1p, g1, dinv, b1r, W2p)
    s2p = _sc_aggregate(src, dst, g2, zeros)
    out = _tc_last_call(s2p, g2, dinv, b2p)
    return out[0:N, 0:A]


# speedup vs baseline: 71.6881x; 1.3387x over previous
import functools

import jax
import jax.numpy as jnp
from jax import lax
from jax.experimental import pallas as pl
from jax.experimental.pallas import tpu as pltpu
from jax.experimental.pallas import tpu_sc as plsc

N = 10000
D = 128
H = 16
A = 8
E = 320000

NTILES = 32
C = 128
K = 8
NPAIR = 5
NITER = 2 * NPAIR
CHUNKS = K * NITER
EP = NTILES * CHUNKS * C
ECHUNKS = EP // C + K
ROWS_PER_TILE = 632
NP = 16 * ROWS_PER_TILE

_mesh = plsc.VectorSubcoreMesh(core_axis_name="c", subcore_axis_name="s")
_sc_params = pltpu.CompilerParams(use_tc_tiling_on_sc=False)


@functools.partial(
    pl.kernel,
    mesh=_mesh,
    compiler_params=_sc_params,
    out_type=jax.ShapeDtypeStruct((2 * NP, H), jnp.float32),
    scratch_types=[
        pltpu.VMEM((K, C), jnp.int32),
        pltpu.VMEM((C, H), jnp.float32),
        pltpu.VMEM_SHARED((NP, H), jnp.float32),
        pltpu.SemaphoreType.DMA,
        pltpu.SemaphoreType.DMA,
    ],
)
def _sc_degree(dst_hbm, ones_hbm, zeros_hbm, out_hbm,
               dst_v, ones_v, acc, isem, ssem):
    c = lax.axis_index("c")
    s = lax.axis_index("s")
    wid = s * 2 + c
    r0 = s * ROWS_PER_TILE
    pltpu.sync_copy(zeros_hbm.at[pl.ds(r0, ROWS_PER_TILE)],
                    acc.at[pl.ds(r0, ROWS_PER_TILE)])
    pltpu.sync_copy(ones_hbm, ones_v)
    plsc.subcore_barrier()
    base = wid * CHUNKS

    def body(i, carry):
        row0 = base + i * K
        pltpu.async_copy(dst_hbm.at[pl.ds(row0, K)], dst_v, isem).wait()
        scat = [pltpu.async_copy(ones_v, acc.at[dst_v.at[j]], ssem, add=True)
                for j in range(K)]
        for d in scat:
            d.wait()
        return carry

    lax.fori_loop(0, NITER, body, 0)
    plsc.subcore_barrier()
    pltpu.sync_copy(acc.at[pl.ds(r0, ROWS_PER_TILE)],
                    out_hbm.at[pl.ds(c * NP + r0, ROWS_PER_TILE)])


@functools.partial(
    pl.kernel,
    mesh=_mesh,
    compiler_params=_sc_params,
    out_type=jax.ShapeDtypeStruct((2 * NP, H), jnp.float32),
    scratch_types=[
        pltpu.VMEM((2, K, C), jnp.int32),
        pltpu.VMEM((2, K, C), jnp.int32),
        pltpu.VMEM((2, K, C, H), jnp.float32),
        pltpu.VMEM_SHARED((NP, H), jnp.float32),
        pltpu.VMEM_SHARED((NP, H), jnp.float32),
        pltpu.SemaphoreType.DMA,
        pltpu.SemaphoreType.DMA,
        pltpu.SemaphoreType.DMA,
        pltpu.SemaphoreType.DMA,
    ],
)
def _sc_aggregate(src_hbm, dst_hbm, g_hbm, zeros_hbm, out_hbm,
                  src_v, dst_v, rows_v, gtab, acc, isem0, isem1, gsem, ssem):
    c = lax.axis_index("c")
    s = lax.axis_index("s")
    wid = s * 2 + c
    r0 = s * ROWS_PER_TILE
    pltpu.sync_copy(zeros_hbm.at[pl.ds(r0, ROWS_PER_TILE)],
                    acc.at[pl.ds(r0, ROWS_PER_TILE)])
    pltpu.sync_copy(g_hbm.at[pl.ds(r0, ROWS_PER_TILE)],
                    gtab.at[pl.ds(r0, ROWS_PER_TILE)])
    plsc.subcore_barrier()
    base = wid * CHUNKS

    def idx_copy(row0, b, sem):
        return (pltpu.async_copy(src_hbm.at[pl.ds(row0, K)], src_v.at[b],
                                 sem),
                pltpu.async_copy(dst_hbm.at[pl.ds(row0, K)], dst_v.at[b],
                                 sem))

    def fire_gathers(b):
        return [pltpu.async_copy(gtab.at[src_v.at[b, j]], rows_v.at[b, j],
                                 gsem)
                for j in range(K)]

    def fire_scatters(b):
        return [pltpu.async_copy(rows_v.at[b, j], acc.at[dst_v.at[b, j]],
                                 ssem, add=True)
                for j in range(K)]

    idx_copy(base, 0, isem0)

    def body(i, carry):
        row0 = base + (2 * i) * K
        i1 = idx_copy(row0 + K, 1, isem1)
        pltpu.make_async_copy(src_hbm.at[pl.ds(row0, K)], src_v.at[0],
                              isem0).wait()
        pltpu.make_async_copy(dst_hbm.at[pl.ds(row0, K)], dst_v.at[0],
                              isem0).wait()
        ga = fire_gathers(0)
        for d in ga:
            d.wait()
        sa = fire_scatters(0)
        i1[0].wait()
        i1[1].wait()
        gb = fire_gathers(1)
        for d in gb:
            d.wait()
        sb = fire_scatters(1)
        for d in sa:
            d.wait()
        idx_copy(row0 + 2 * K, 0, isem0)
        for d in sb:
            d.wait()
        return carry

    lax.fori_loop(0, NPAIR, body, 0)
    pltpu.make_async_copy(src_hbm.at[pl.ds(base + CHUNKS, K)], src_v.at[0],
                          isem0).wait()
    pltpu.make_async_copy(dst_hbm.at[pl.ds(base + CHUNKS, K)], dst_v.at[0],
                          isem0).wait()
    plsc.subcore_barrier()
    pltpu.sync_copy(acc.at[pl.ds(r0, ROWS_PER_TILE)],
                    out_hbm.at[pl.ds(c * NP + r0, ROWS_PER_TILE)])


NPq = NP // 8
Nq = N // 8


def _tc_first(xq_ref, w1b_ref, degp_ref, g1_ref, dinv_ref):
    deg = degp_ref[0:NPq, :] + degp_ref[NPq:2 * NPq, :] + 1.0
    dinv = lax.rsqrt(deg)
    dinv_ref[...] = dinv
    h1p = jnp.dot(xq_ref[...], w1b_ref[...],
                  preferred_element_type=jnp.float32)
    g1_ref[0:Nq, :] = h1p * dinv[0:Nq, :]
    g1_ref[Nq:NPq, :] = jnp.zeros((NPq - Nq, 8 * H), jnp.float32)


def _tc_mid(s1p_ref, g1_ref, dinv_ref, b1_ref, w2b_ref, g2_ref):
    s = s1p_ref[0:NPq, :] + s1p_ref[NPq:2 * NPq, :] + g1_ref[...]
    h = jnp.maximum(s * dinv_ref[...] + b1_ref[...], 0.0)
    h2 = jnp.dot(h, w2b_ref[...], preferred_element_type=jnp.float32)
    g2_ref[...] = h2 * dinv_ref[...]


def _tc_last(s2p_ref, g2_ref, dinv_ref, b2_ref, out_ref):
    s = s2p_ref[0:NPq, :] + s2p_ref[NPq:2 * NPq, :] + g2_ref[...]
    out_ref[...] = s * dinv_ref[...] + b2_ref[...]


_tc_first_call = pl.pallas_call(
    _tc_first,
    out_shape=(jax.ShapeDtypeStruct((NPq, 8 * H), jnp.float32),
               jax.ShapeDtypeStruct((NPq, 8 * H), jnp.float32)),
)

_tc_mid_call = pl.pallas_call(
    _tc_mid,
    out_shape=jax.ShapeDtypeStruct((NPq, 8 * H), jnp.float32),
)

_tc_last_call = pl.pallas_call(
    _tc_last,
    out_shape=jax.ShapeDtypeStruct((NPq, 8 * H), jnp.float32),
)


def kernel(x, edge_index, W1, b1, W2, b2):
    pad = ECHUNKS * C - E
    src = jnp.concatenate([edge_index[0],
                           jnp.full((pad,), N, jnp.int32)]).reshape(ECHUNKS, C)
    dst = jnp.concatenate([edge_index[1],
                           jnp.full((pad,), N, jnp.int32)]).reshape(ECHUNKS, C)
    zeros = jnp.zeros((NP, H), jnp.float32)
    ones = jnp.ones((C, H), jnp.float32)
    W2p = jnp.pad(W2, ((0, 0), (0, H - A)))
    eye8 = jnp.eye(8, dtype=jnp.float32)
    w1b = (eye8[:, None, :, None]
           * W1[None, :, None, :]).reshape(8 * D, 8 * H)
    w2b = (eye8[:, None, :, None]
           * W2p[None, :, None, :]).reshape(8 * H, 8 * H)
    b1p = jnp.tile(b1, 8).reshape(1, 8 * H)
    b2p = jnp.tile(jnp.pad(b2, (0, H - A)), 8).reshape(1, 8 * H)

    degp = _sc_degree(dst, ones, zeros).reshape(2 * NPq, 8 * H)
    g1p, dinvp = _tc_first_call(x.reshape(Nq, 8 * D), w1b, degp)
    s1p = _sc_aggregate(src, dst, g1p.reshape(NP, H),
                        zeros).reshape(2 * NPq, 8 * H)
    g2p = _tc_mid_call(s1p, g1p, dinvp, b1p, w2b)
    s2p = _sc_aggregate(src, dst, g2p.reshape(NP, H),
                        zeros).reshape(2 * NPq, 8 * H)
    outp = _tc_last_call(s2p, g2p, dinvp, b2p)
    return outp.reshape(NP, H)[0:N, 0:A]


# speedup vs baseline: 79.7874x; 1.1130x over previous
import functools

import jax
import jax.numpy as jnp
from jax import lax
from jax.experimental import pallas as pl
from jax.experimental.pallas import tpu as pltpu
from jax.experimental.pallas import tpu_sc as plsc

N = 10000
D = 128
H = 16
A = 8
E = 320000

NTILES = 32
C = 128
K = 8
NPAIR = 5
NITER = 2 * NPAIR
CHUNKS = K * NITER
EP = NTILES * CHUNKS * C
ECHUNKS = EP // C + K
ROWS_PER_TILE = 632
NP = 16 * ROWS_PER_TILE
NPq = NP // 8
QROWS = ROWS_PER_TILE // 8

_mesh = plsc.VectorSubcoreMesh(core_axis_name="c", subcore_axis_name="s")
_sc_params = pltpu.CompilerParams(use_tc_tiling_on_sc=False)


@functools.partial(
    pl.kernel,
    mesh=_mesh,
    compiler_params=_sc_params,
    out_type=jax.ShapeDtypeStruct((2 * NP, H), jnp.float32),
    scratch_types=[
        pltpu.VMEM((K, C), jnp.int32),
        pltpu.VMEM((C, H), jnp.float32),
        pltpu.VMEM_SHARED((NP, H), jnp.float32),
        pltpu.SemaphoreType.DMA,
        pltpu.SemaphoreType.DMA,
    ],
)
def _sc_degree(ei_hbm, ones_hbm, zeros_hbm, out_hbm,
               dst_v, ones_v, acc, isem, ssem):
    c = lax.axis_index("c")
    s = lax.axis_index("s")
    wid = s * 2 + c
    r0 = s * ROWS_PER_TILE
    pltpu.sync_copy(zeros_hbm.at[pl.ds(r0, ROWS_PER_TILE)],
                    acc.at[pl.ds(r0, ROWS_PER_TILE)])
    pltpu.sync_copy(ones_hbm, ones_v)
    plsc.subcore_barrier()
    base = wid * CHUNKS

    def body(i, carry):
        row0 = base + i * K
        pltpu.async_copy(ei_hbm.at[1, pl.ds(row0, K)], dst_v, isem).wait()
        scat = [pltpu.async_copy(ones_v, acc.at[dst_v.at[j]], ssem, add=True)
                for j in range(K)]
        for d in scat:
            d.wait()
        return carry

    lax.fori_loop(0, NITER, body, 0)
    plsc.subcore_barrier()
    pltpu.sync_copy(acc.at[pl.ds(r0, ROWS_PER_TILE)],
                    out_hbm.at[pl.ds(c * NP + r0, ROWS_PER_TILE)])


@functools.partial(
    pl.kernel,
    mesh=_mesh,
    compiler_params=_sc_params,
    out_type=jax.ShapeDtypeStruct((2 * NP, H), jnp.float32),
    scratch_types=[
        pltpu.VMEM((2, K, C), jnp.int32),
        pltpu.VMEM((2, K, C), jnp.int32),
        pltpu.VMEM((2, K, C, H), jnp.float32),
        pltpu.VMEM_SHARED((NP, H), jnp.float32),
        pltpu.VMEM_SHARED((NP, H), jnp.float32),
        pltpu.SemaphoreType.DMA,
        pltpu.SemaphoreType.DMA,
        pltpu.SemaphoreType.DMA,
        pltpu.SemaphoreType.DMA,
    ],
)
def _sc_aggregate(ei_hbm, g_hbm, zeros_hbm, out_hbm,
                  src_v, dst_v, rows_v, gtab, acc, isem0, isem1, gsem, ssem):
    c = lax.axis_index("c")
    s = lax.axis_index("s")
    wid = s * 2 + c
    r0 = s * ROWS_PER_TILE
    pltpu.sync_copy(zeros_hbm.at[pl.ds(r0, ROWS_PER_TILE)],
                    acc.at[pl.ds(r0, ROWS_PER_TILE)])
    pltpu.sync_copy(g_hbm.at[pl.ds(r0, ROWS_PER_TILE)],
                    gtab.at[pl.ds(r0, ROWS_PER_TILE)])
    plsc.subcore_barrier()
    base = wid * CHUNKS

    def idx_copy(row0, b, sem):
        return (pltpu.async_copy(ei_hbm.at[0, pl.ds(row0, K)], src_v.at[b],
                                 sem),
                pltpu.async_copy(ei_hbm.at[1, pl.ds(row0, K)], dst_v.at[b],
                                 sem))

    def fire_gathers(b):
        return [pltpu.async_copy(gtab.at[src_v.at[b, j]], rows_v.at[b, j],
                                 gsem)
                for j in range(K)]

    def fire_scatters(b):
        return [pltpu.async_copy(rows_v.at[b, j], acc.at[dst_v.at[b, j]],
                                 ssem, add=True)
                for j in range(K)]

    idx_copy(base, 0, isem0)

    def body(i, carry):
        row0 = base + (2 * i) * K
        i1 = idx_copy(row0 + K, 1, isem1)
        pltpu.make_async_copy(ei_hbm.at[0, pl.ds(row0, K)], src_v.at[0],
                              isem0).wait()
        pltpu.make_async_copy(ei_hbm.at[1, pl.ds(row0, K)], dst_v.at[0],
                              isem0).wait()
        ga = fire_gathers(0)
        for d in ga:
            d.wait()
        sa = fire_scatters(0)
        i1[0].wait()
        i1[1].wait()
        gb = fire_gathers(1)
        for d in gb:
            d.wait()
        sb = fire_scatters(1)
        for d in sa:
            d.wait()
        idx_copy(row0 + 2 * K, 0, isem0)
        for d in sb:
            d.wait()
        return carry

    lax.fori_loop(0, NPAIR, body, 0)
    pltpu.make_async_copy(ei_hbm.at[0, pl.ds(base + CHUNKS, K)], src_v.at[0],
                          isem0).wait()
    pltpu.make_async_copy(ei_hbm.at[1, pl.ds(base + CHUNKS, K)], dst_v.at[0],
                          isem0).wait()
    plsc.subcore_barrier()
    pltpu.sync_copy(acc.at[pl.ds(r0, ROWS_PER_TILE)],
                    out_hbm.at[pl.ds(c * NP + r0, ROWS_PER_TILE)])


Nq = N // 8


def _tc_first(xq_ref, w1b_ref, degp_ref, g1_ref, dinv_ref):
    deg = degp_ref[0:NPq, :] + degp_ref[NPq:2 * NPq, :] + 1.0
    dinv = lax.rsqrt(deg)
    dinv_ref[...] = dinv
    h1p = jnp.dot(xq_ref[...], w1b_ref[...],
                  preferred_element_type=jnp.float32)
    g1_ref[0:Nq, :] = h1p * dinv[0:Nq, :]
    g1_ref[Nq:NPq, :] = jnp.zeros((NPq - Nq, 8 * H), jnp.float32)


def _tc_mid(s1p_ref, g1_ref, dinv_ref, b1_ref, w2b_ref, g2_ref):
    s = s1p_ref[0:NPq, :] + s1p_ref[NPq:2 * NPq, :] + g1_ref[...]
    h = jnp.maximum(s * dinv_ref[...] + b1_ref[...], 0.0)
    h2 = jnp.dot(h, w2b_ref[...], preferred_element_type=jnp.float32)
    g2_ref[...] = h2 * dinv_ref[...]


def _tc_last(s2p_ref, g2_ref, dinv_ref, b2_ref, out_ref):
    s = s2p_ref[0:NPq, :] + s2p_ref[NPq:2 * NPq, :] + g2_ref[...]
    out_ref[...] = s * dinv_ref[...] + b2_ref[...]


_tc_first_call = pl.pallas_call(
    _tc_first,
    out_shape=(jax.ShapeDtypeStruct((NPq, 8 * H), jnp.float32),
               jax.ShapeDtypeStruct((NPq, 8 * H), jnp.float32)),
)

_tc_mid_call = pl.pallas_call(
    _tc_mid,
    out_shape=jax.ShapeDtypeStruct((NPq, 8 * H), jnp.float32),
)

_tc_last_call = pl.pallas_call(
    _tc_last,
    out_shape=jax.ShapeDtypeStruct((NPq, 8 * H), jnp.float32),
)


def kernel(x, edge_index, W1, b1, W2, b2):
    ei = jnp.pad(edge_index.reshape(2, E // C, C),
                 ((0, 0), (0, ECHUNKS - E // C), (0, 0)),
                 constant_values=N)
    zeros = jnp.zeros((NP, H), jnp.float32)
    ones = jnp.ones((C, H), jnp.float32)
    W2p = jnp.pad(W2, ((0, 0), (0, H - A)))
    eye8 = jnp.eye(8, dtype=jnp.float32)
    w1b = (eye8[:, None, :, None]
           * W1[None, :, None, :]).reshape(8 * D, 8 * H)
    w2b = (eye8[:, None, :, None]
           * W2p[None, :, None, :]).reshape(8 * H, 8 * H)
    b1p = jnp.tile(b1, 8).reshape(1, 8 * H)
    b2p = jnp.tile(jnp.pad(b2, (0, H - A)), 8).reshape(1, 8 * H)

    degp = _sc_degree(ei, ones, zeros).reshape(2 * NPq, 8 * H)
    g1p, dinvp = _tc_first_call(x.reshape(Nq, 8 * D), w1b, degp)
    s1p = _sc_aggregate(ei, g1p.reshape(NP, H),
                        zeros).reshape(2 * NPq, 8 * H)
    g2p = _tc_mid_call(s1p, g1p, dinvp, b1p, w2b)
    s2p = _sc_aggregate(ei, g2p.reshape(NP, H),
                        zeros).reshape(2 * NPq, 8 * H)
    outp = _tc_last_call(s2p, g2p, dinvp, b2p)
    return outp[0:Nq].reshape(Nq, 8, H)[:, :, 0:A].reshape(N, A)


# speedup vs baseline: 81.2184x; 1.0179x over previous
import functools

import jax
import jax.numpy as jnp
from jax import lax
from jax.experimental import pallas as pl
from jax.experimental.pallas import tpu as pltpu
from jax.experimental.pallas import tpu_sc as plsc

N = 10000
D = 128
H = 16
A = 8
E = 320000

NTILES = 32
C = 128
K = 8
NPAIR = 5
NITER = 2 * NPAIR
CHUNKS = K * NITER
EP = NTILES * CHUNKS * C
ECHUNKS = EP // C + K
ROWS_PER_TILE = 632
NP = 16 * ROWS_PER_TILE
NPq = NP // 8
QROWS = ROWS_PER_TILE // 8

_mesh = plsc.VectorSubcoreMesh(core_axis_name="c", subcore_axis_name="s")
_sc_params = pltpu.CompilerParams(use_tc_tiling_on_sc=False)


@functools.partial(
    pl.kernel,
    mesh=_mesh,
    compiler_params=_sc_params,
    out_type=jax.ShapeDtypeStruct((2 * NP, H), jnp.float32),
    scratch_types=[
        pltpu.VMEM((2, K, C), jnp.int32),
        pltpu.VMEM((C, H), jnp.float32),
        pltpu.VMEM_SHARED((NP, H), jnp.float32),
        pltpu.SemaphoreType.DMA,
        pltpu.SemaphoreType.DMA,
        pltpu.SemaphoreType.DMA,
    ],
)
def _sc_degree(ei_hbm, ones_hbm, zeros_hbm, out_hbm,
               dst_v, ones_v, acc, isem0, isem1, ssem):
    c = lax.axis_index("c")
    s = lax.axis_index("s")
    wid = s * 2 + c
    r0 = s * ROWS_PER_TILE
    pltpu.sync_copy(zeros_hbm.at[pl.ds(r0, ROWS_PER_TILE)],
                    acc.at[pl.ds(r0, ROWS_PER_TILE)])
    pltpu.sync_copy(ones_hbm, ones_v)
    plsc.subcore_barrier()
    base = wid * CHUNKS

    def idx_copy(row0, b, sem):
        return pltpu.async_copy(ei_hbm.at[1, pl.ds(row0, K)], dst_v.at[b],
                                sem)

    def fire_scatters(b):
        return [pltpu.async_copy(ones_v, acc.at[dst_v.at[b, j]], ssem,
                                 add=True)
                for j in range(K)]

    idx_copy(base, 0, isem0)

    def body(i, carry):
        row0 = base + (2 * i) * K
        i1 = idx_copy(row0 + K, 1, isem1)
        pltpu.make_async_copy(ei_hbm.at[1, pl.ds(row0, K)], dst_v.at[0],
                              isem0).wait()
        sa = fire_scatters(0)
        i1.wait()
        sb = fire_scatters(1)
        for d in sa:
            d.wait()
        idx_copy(row0 + 2 * K, 0, isem0)
        for d in sb:
            d.wait()
        return carry

    lax.fori_loop(0, NPAIR, body, 0)
    pltpu.make_async_copy(ei_hbm.at[1, pl.ds(base + CHUNKS, K)], dst_v.at[0],
                          isem0).wait()
    plsc.subcore_barrier()
    pltpu.sync_copy(acc.at[pl.ds(r0, ROWS_PER_TILE)],
                    out_hbm.at[pl.ds(c * NP + r0, ROWS_PER_TILE)])


@functools.partial(
    pl.kernel,
    mesh=_mesh,
    compiler_params=_sc_params,
    out_type=jax.ShapeDtypeStruct((2 * NP, H), jnp.float32),
    scratch_types=[
        pltpu.VMEM((2, K, C), jnp.int32),
        pltpu.VMEM((2, K, C), jnp.int32),
        pltpu.VMEM((2, K, C, H), jnp.float32),
        pltpu.VMEM_SHARED((NP, H), jnp.float32),
        pltpu.VMEM_SHARED((NP, H), jnp.float32),
        pltpu.SemaphoreType.DMA,
        pltpu.SemaphoreType.DMA,
        pltpu.SemaphoreType.DMA,
        pltpu.SemaphoreType.DMA,
    ],
)
def _sc_aggregate(ei_hbm, g_hbm, zeros_hbm, out_hbm,
                  src_v, dst_v, rows_v, gtab, acc, isem0, isem1, gsem, ssem):
    c = lax.axis_index("c")
    s = lax.axis_index("s")
    wid = s * 2 + c
    r0 = s * ROWS_PER_TILE
    pltpu.sync_copy(zeros_hbm.at[pl.ds(r0, ROWS_PER_TILE)],
                    acc.at[pl.ds(r0, ROWS_PER_TILE)])
    pltpu.sync_copy(g_hbm.at[pl.ds(r0, ROWS_PER_TILE)],
                    gtab.at[pl.ds(r0, ROWS_PER_TILE)])
    plsc.subcore_barrier()
    base = wid * CHUNKS

    def idx_copy(row0, b, sem):
        return (pltpu.async_copy(ei_hbm.at[0, pl.ds(row0, K)], src_v.at[b],
                                 sem),
                pltpu.async_copy(ei_hbm.at[1, pl.ds(row0, K)], dst_v.at[b],
                                 sem))

    def fire_gathers(b):
        return [pltpu.async_copy(gtab.at[src_v.at[b, j]], rows_v.at[b, j],
                                 gsem)
                for j in range(K)]

    def fire_scatters(b):
        return [pltpu.async_copy(rows_v.at[b, j], acc.at[dst_v.at[b, j]],
                                 ssem, add=True)
                for j in range(K)]

    idx_copy(base, 0, isem0)

    def body(i, carry):
        row0 = base + (2 * i) * K
        i1 = idx_copy(row0 + K, 1, isem1)
        pltpu.make_async_copy(ei_hbm.at[0, pl.ds(row0, K)], src_v.at[0],
                              isem0).wait()
        pltpu.make_async_copy(ei_hbm.at[1, pl.ds(row0, K)], dst_v.at[0],
                              isem0).wait()
        ga = fire_gathers(0)
        for d in ga:
            d.wait()
        sa = fire_scatters(0)
        i1[0].wait()
        i1[1].wait()
        gb = fire_gathers(1)
        for d in gb:
            d.wait()
        sb = fire_scatters(1)
        for d in sa:
            d.wait()
        idx_copy(row0 + 2 * K, 0, isem0)
        for d in sb:
            d.wait()
        return carry

    lax.fori_loop(0, NPAIR, body, 0)
    pltpu.make_async_copy(ei_hbm.at[0, pl.ds(base + CHUNKS, K)], src_v.at[0],
                          isem0).wait()
    pltpu.make_async_copy(ei_hbm.at[1, pl.ds(base + CHUNKS, K)], dst_v.at[0],
                          isem0).wait()
    plsc.subcore_barrier()
    pltpu.sync_copy(acc.at[pl.ds(r0, ROWS_PER_TILE)],
                    out_hbm.at[pl.ds(c * NP + r0, ROWS_PER_TILE)])


Nq = N // 8


def _tc_first(xq_ref, w1b_ref, degp_ref, g1_ref, dinv_ref):
    deg = degp_ref[0:NPq, :] + degp_ref[NPq:2 * NPq, :] + 1.0
    dinv = lax.rsqrt(deg)
    dinv_ref[...] = dinv
    h1p = jnp.dot(xq_ref[...], w1b_ref[...],
                  preferred_element_type=jnp.float32)
    g1_ref[0:Nq, :] = h1p * dinv[0:Nq, :]
    g1_ref[Nq:NPq, :] = jnp.zeros((NPq - Nq, 8 * H), jnp.float32)


def _tc_mid(s1p_ref, g1_ref, dinv_ref, b1_ref, w2b_ref, g2_ref):
    s = s1p_ref[0:NPq, :] + s1p_ref[NPq:2 * NPq, :] + g1_ref[...]
    h = jnp.maximum(s * dinv_ref[...] + b1_ref[...], 0.0)
    h2 = jnp.dot(h, w2b_ref[...], preferred_element_type=jnp.float32)
    g2_ref[...] = h2 * dinv_ref[...]


def _tc_last(s2p_ref, g2_ref, dinv_ref, b2_ref, out_ref):
    s = s2p_ref[0:NPq, :] + s2p_ref[NPq:2 * NPq, :] + g2_ref[...]
    out_ref[...] = s * dinv_ref[...] + b2_ref[...]


_tc_first_call = pl.pallas_call(
    _tc_first,
    out_shape=(jax.ShapeDtypeStruct((NPq, 8 * H), jnp.float32),
               jax.ShapeDtypeStruct((NPq, 8 * H), jnp.float32)),
)

_tc_mid_call = pl.pallas_call(
    _tc_mid,
    out_shape=jax.ShapeDtypeStruct((NPq, 8 * H), jnp.float32),
)

_tc_last_call = pl.pallas_call(
    _tc_last,
    out_shape=jax.ShapeDtypeStruct((NPq, 8 * H), jnp.float32),
)


def kernel(x, edge_index, W1, b1, W2, b2):
    ei = jnp.pad(edge_index.reshape(2, E // C, C),
                 ((0, 0), (0, ECHUNKS - E // C), (0, 0)),
                 constant_values=N)
    zeros = jnp.zeros((NP, H), jnp.float32)
    ones = jnp.ones((C, H), jnp.float32)
    W2p = jnp.pad(W2, ((0, 0), (0, H - A)))
    eye8 = jnp.eye(8, dtype=jnp.float32)
    w1b = (eye8[:, None, :, None]
           * W1[None, :, None, :]).reshape(8 * D, 8 * H)
    w2b = (eye8[:, None, :, None]
           * W2p[None, :, None, :]).reshape(8 * H, 8 * H)
    b1p = jnp.tile(b1, 8).reshape(1, 8 * H)
    b2p = jnp.tile(jnp.pad(b2, (0, H - A)), 8).reshape(1, 8 * H)

    degp = _sc_degree(ei, ones, zeros).reshape(2 * NPq, 8 * H)
    g1p, dinvp = _tc_first_call(x.reshape(Nq, 8 * D), w1b, degp)
    s1p = _sc_aggregate(ei, g1p.reshape(NP, H),
                        zeros).reshape(2 * NPq, 8 * H)
    g2p = _tc_mid_call(s1p, g1p, dinvp, b1p, w2b)
    s2p = _sc_aggregate(ei, g2p.reshape(NP, H),
                        zeros).reshape(2 * NPq, 8 * H)
    outp = _tc_last_call(s2p, g2p, dinvp, b2p)
    return outp[0:Nq].reshape(Nq, 8, H)[:, :, 0:A].reshape(N, A)


# speedup vs baseline: 81.5540x; 1.0041x over previous
import functools

import jax
import jax.numpy as jnp
from jax import lax
from jax.experimental import pallas as pl
from jax.experimental.pallas import tpu as pltpu
from jax.experimental.pallas import tpu_sc as plsc

N = 10000
D = 128
H = 16
A = 8
E = 320000

NTILES = 32
C = 128
K = 8
NPAIR = 5
NITER = 2 * NPAIR
CHUNKS = K * NITER
EP = NTILES * CHUNKS * C
ECHUNKS = EP // C + K
ROWS_PER_TILE = 632
NP = 16 * ROWS_PER_TILE
NPq = NP // 8
QROWS = ROWS_PER_TILE // 8

_mesh = plsc.VectorSubcoreMesh(core_axis_name="c", subcore_axis_name="s")
_sc_params = pltpu.CompilerParams(use_tc_tiling_on_sc=False)


@functools.partial(
    pl.kernel,
    mesh=_mesh,
    compiler_params=_sc_params,
    out_type=jax.ShapeDtypeStruct((2 * NP, H), jnp.float32),
    scratch_types=[
        pltpu.VMEM((2, K, C), jnp.int32),
        pltpu.VMEM((C, H), jnp.float32),
        pltpu.VMEM_SHARED((NP, H), jnp.float32),
        pltpu.SemaphoreType.DMA,
        pltpu.SemaphoreType.DMA,
        pltpu.SemaphoreType.DMA,
    ],
)
def _sc_degree(ei_hbm, ones_hbm, zeros_hbm, out_hbm,
               dst_v, ones_v, acc, isem0, isem1, ssem):
    c = lax.axis_index("c")
    s = lax.axis_index("s")
    wid = s * 2 + c
    r0 = s * ROWS_PER_TILE
    pltpu.sync_copy(zeros_hbm.at[pl.ds(r0, ROWS_PER_TILE)],
                    acc.at[pl.ds(r0, ROWS_PER_TILE)])
    pltpu.sync_copy(ones_hbm, ones_v)
    plsc.subcore_barrier()
    base = wid * CHUNKS

    def idx_copy(row0, b, sem):
        return pltpu.async_copy(ei_hbm.at[1, pl.ds(row0, K)], dst_v.at[b],
                                sem)

    def fire_scatters(b):
        return [pltpu.async_copy(ones_v, acc.at[dst_v.at[b, j]], ssem,
                                 add=True)
                for j in range(K)]

    idx_copy(base, 0, isem0)

    def body(i, carry):
        row0 = base + (2 * i) * K
        i1 = idx_copy(row0 + K, 1, isem1)
        pltpu.make_async_copy(ei_hbm.at[1, pl.ds(row0, K)], dst_v.at[0],
                              isem0).wait()
        sa = fire_scatters(0)
        i1.wait()
        sb = fire_scatters(1)
        for d in sa:
            d.wait()
        idx_copy(row0 + 2 * K, 0, isem0)
        for d in sb:
            d.wait()
        return carry

    lax.fori_loop(0, NPAIR, body, 0)
    pltpu.make_async_copy(ei_hbm.at[1, pl.ds(base + CHUNKS, K)], dst_v.at[0],
                          isem0).wait()
    plsc.subcore_barrier()
    pltpu.sync_copy(acc.at[pl.ds(r0, ROWS_PER_TILE)],
                    out_hbm.at[pl.ds(c * NP + r0, ROWS_PER_TILE)])


@functools.partial(
    pl.kernel,
    mesh=_mesh,
    compiler_params=_sc_params,
    out_type=jax.ShapeDtypeStruct((2 * NP, H), jnp.float32),
    scratch_types=[
        pltpu.VMEM((2, K, C), jnp.int32),
        pltpu.VMEM((2, K, C), jnp.int32),
        pltpu.VMEM((2, K, C, H), jnp.float32),
        pltpu.VMEM_SHARED((NP, H), jnp.float32),
        pltpu.VMEM_SHARED((NP, H), jnp.float32),
        pltpu.SemaphoreType.DMA,
        pltpu.SemaphoreType.DMA,
        pltpu.SemaphoreType.DMA,
        pltpu.SemaphoreType.DMA,
    ],
)
def _sc_aggregate(ei_hbm, g_hbm, zeros_hbm, out_hbm,
                  src_v, dst_v, rows_v, gtab, acc, isem0, isem1, gsem, ssem):
    c = lax.axis_index("c")
    s = lax.axis_index("s")
    wid = s * 2 + c
    r0 = s * ROWS_PER_TILE
    pltpu.sync_copy(zeros_hbm.at[pl.ds(r0, ROWS_PER_TILE)],
                    acc.at[pl.ds(r0, ROWS_PER_TILE)])
    pltpu.sync_copy(g_hbm.at[pl.ds(r0, ROWS_PER_TILE)],
                    gtab.at[pl.ds(r0, ROWS_PER_TILE)])
    plsc.subcore_barrier()
    base = wid * CHUNKS

    def idx_copy(row0, b, sem):
        return (pltpu.async_copy(ei_hbm.at[0, pl.ds(row0, K)], src_v.at[b],
                                 sem),
                pltpu.async_copy(ei_hbm.at[1, pl.ds(row0, K)], dst_v.at[b],
                                 sem))

    def fire_gathers(b):
        return [pltpu.async_copy(gtab.at[src_v.at[b, j]], rows_v.at[b, j],
                                 gsem)
                for j in range(K)]

    def fire_scatters(b):
        return [pltpu.async_copy(rows_v.at[b, j], acc.at[dst_v.at[b, j]],
                                 ssem, add=True)
                for j in range(K)]

    idx_copy(base, 0, isem0)

    def body(i, carry):
        row0 = base + (2 * i) * K
        i1 = idx_copy(row0 + K, 1, isem1)
        pltpu.make_async_copy(ei_hbm.at[0, pl.ds(row0, K)], src_v.at[0],
                              isem0).wait()
        pltpu.make_async_copy(ei_hbm.at[1, pl.ds(row0, K)], dst_v.at[0],
                              isem0).wait()
        ga = fire_gathers(0)
        for d in ga:
            d.wait()
        sa = fire_scatters(0)
        i1[0].wait()
        i1[1].wait()
        gb = fire_gathers(1)
        for d in gb:
            d.wait()
        sb = fire_scatters(1)
        for d in sa:
            d.wait()
        idx_copy(row0 + 2 * K, 0, isem0)
        for d in sb:
            d.wait()
        return carry

    lax.fori_loop(0, NPAIR, body, 0)
    pltpu.make_async_copy(ei_hbm.at[0, pl.ds(base + CHUNKS, K)], src_v.at[0],
                          isem0).wait()
    pltpu.make_async_copy(ei_hbm.at[1, pl.ds(base + CHUNKS, K)], dst_v.at[0],
                          isem0).wait()
    plsc.subcore_barrier()
    pltpu.sync_copy(acc.at[pl.ds(r0, ROWS_PER_TILE)],
                    out_hbm.at[pl.ds(c * NP + r0, ROWS_PER_TILE)])


Nq = N // 8


def _tc_matmul1(xq_ref, w1b_ref, h1p_ref):
    h1p_ref[...] = jnp.dot(xq_ref[...], w1b_ref[...],
                           preferred_element_type=jnp.float32)


def _tc_first(h1p_ref, degp_ref, g1_ref, dinv_ref):
    deg = degp_ref[0:NPq, :] + degp_ref[NPq:2 * NPq, :] + 1.0
    dinv = lax.rsqrt(deg)
    dinv_ref[...] = dinv
    g1_ref[0:Nq, :] = h1p_ref[...] * dinv[0:Nq, :]
    g1_ref[Nq:NPq, :] = jnp.zeros((NPq - Nq, 8 * H), jnp.float32)


def _tc_mid(s1p_ref, g1_ref, dinv_ref, b1_ref, w2b_ref, g2_ref):
    s = s1p_ref[0:NPq, :] + s1p_ref[NPq:2 * NPq, :] + g1_ref[...]
    h = jnp.maximum(s * dinv_ref[...] + b1_ref[...], 0.0)
    h2 = jnp.dot(h, w2b_ref[...], preferred_element_type=jnp.float32)
    g2_ref[...] = h2 * dinv_ref[...]


def _tc_last(s2p_ref, g2_ref, dinv_ref, b2_ref, out_ref):
    s = s2p_ref[0:NPq, :] + s2p_ref[NPq:2 * NPq, :] + g2_ref[...]
    out_ref[...] = s * dinv_ref[...] + b2_ref[...]


_tc_matmul1_call = pl.pallas_call(
    _tc_matmul1,
    out_shape=jax.ShapeDtypeStruct((Nq, 8 * H), jnp.float32),
)

_tc_first_call = pl.pallas_call(
    _tc_first,
    out_shape=(jax.ShapeDtypeStruct((NPq, 8 * H), jnp.float32),
               jax.ShapeDtypeStruct((NPq, 8 * H), jnp.float32)),
)

_tc_mid_call = pl.pallas_call(
    _tc_mid,
    out_shape=jax.ShapeDtypeStruct((NPq, 8 * H), jnp.float32),
)

_tc_last_call = pl.pallas_call(
    _tc_last,
    out_shape=jax.ShapeDtypeStruct((NPq, 8 * H), jnp.float32),
)


def kernel(x, edge_index, W1, b1, W2, b2):
    ei = jnp.pad(edge_index.reshape(2, E // C, C),
                 ((0, 0), (0, ECHUNKS - E // C), (0, 0)),
                 constant_values=N)
    zeros = jnp.zeros((NP, H), jnp.float32)
    ones = jnp.ones((C, H), jnp.float32)
    W2p = jnp.pad(W2, ((0, 0), (0, H - A)))
    eye8 = jnp.eye(8, dtype=jnp.float32)
    w1b = (eye8[:, None, :, None]
           * W1[None, :, None, :]).reshape(8 * D, 8 * H)
    w2b = (eye8[:, None, :, None]
           * W2p[None, :, None, :]).reshape(8 * H, 8 * H)
    b1p = jnp.tile(b1, 8).reshape(1, 8 * H)
    b2p = jnp.tile(jnp.pad(b2, (0, H - A)), 8).reshape(1, 8 * H)

    h1p = _tc_matmul1_call(x.reshape(Nq, 8 * D), w1b)
    degp = _sc_degree(ei, ones, zeros).reshape(2 * NPq, 8 * H)
    g1p, dinvp = _tc_first_call(h1p, degp)
    s1p = _sc_aggregate(ei, g1p.reshape(NP, H),
                        zeros).reshape(2 * NPq, 8 * H)
    g2p = _tc_mid_call(s1p, g1p, dinvp, b1p, w2b)
    s2p = _sc_aggregate(ei, g2p.reshape(NP, H),
                        zeros).reshape(2 * NPq, 8 * H)
    outp = _tc_last_call(s2p, g2p, dinvp, b2p)
    return outp[0:Nq].reshape(Nq, 8, H)[:, :, 0:A].reshape(N, A)


# speedup vs baseline: 81.6248x; 1.0009x over previous
import functools

import jax
import jax.numpy as jnp
from jax import lax
from jax.experimental import pallas as pl
from jax.experimental.pallas import tpu as pltpu
from jax.experimental.pallas import tpu_sc as plsc

N = 10000
D = 128
H = 16
A = 8
E = 320000

NTILES = 32
C = 128
K = 8
NPAIR = 5
NITER = 2 * NPAIR
CHUNKS = K * NITER
EP = NTILES * CHUNKS * C
ECHUNKS = EP // C + K
ROWS_PER_TILE = 632
NP = 16 * ROWS_PER_TILE
NPq = NP // 8

_mesh = plsc.VectorSubcoreMesh(core_axis_name="c", subcore_axis_name="s")
_sc_params = pltpu.CompilerParams(use_tc_tiling_on_sc=False)


@functools.partial(
    pl.kernel,
    mesh=_mesh,
    compiler_params=_sc_params,
    out_type=jax.ShapeDtypeStruct((2 * NP, H), jnp.float32),
    scratch_types=[
        pltpu.VMEM((2, K, C), jnp.int32),
        pltpu.VMEM((C, H), jnp.float32),
        pltpu.VMEM_SHARED((NP, H), jnp.float32),
        pltpu.SemaphoreType.DMA,
        pltpu.SemaphoreType.DMA,
        pltpu.SemaphoreType.DMA,
    ],
)
def _sc_degree(ei_hbm, ones_hbm, zeros_hbm, out_hbm,
               dst_v, ones_v, acc, isem0, isem1, ssem):
    c = lax.axis_index("c")
    s = lax.axis_index("s")
    wid = s * 2 + c
    r0 = s * ROWS_PER_TILE
    pltpu.sync_copy(zeros_hbm.at[pl.ds(r0, ROWS_PER_TILE)],
                    acc.at[pl.ds(r0, ROWS_PER_TILE)])
    pltpu.sync_copy(ones_hbm, ones_v)
    plsc.subcore_barrier()
    base = wid * CHUNKS

    def idx_copy(row0, b, sem):
        return pltpu.async_copy(ei_hbm.at[1, pl.ds(row0, K)], dst_v.at[b],
                                sem)

    def fire_scatters(b):
        return [pltpu.async_copy(ones_v, acc.at[dst_v.at[b, j]], ssem,
                                 add=True)
                for j in range(K)]

    idx_copy(base, 0, isem0)

    def body(i, carry):
        row0 = base + (2 * i) * K
        i1 = idx_copy(row0 + K, 1, isem1)
        pltpu.make_async_copy(ei_hbm.at[1, pl.ds(row0, K)], dst_v.at[0],
                              isem0).wait()
        sa = fire_scatters(0)
        i1.wait()
        sb = fire_scatters(1)
        for d in sa:
            d.wait()
        idx_copy(row0 + 2 * K, 0, isem0)
        for d in sb:
            d.wait()
        return carry

    lax.fori_loop(0, NPAIR, body, 0)
    pltpu.make_async_copy(ei_hbm.at[1, pl.ds(base + CHUNKS, K)], dst_v.at[0],
                          isem0).wait()
    plsc.subcore_barrier()
    pltpu.sync_copy(acc.at[pl.ds(r0, ROWS_PER_TILE)],
                    out_hbm.at[pl.ds(c * NP + r0, ROWS_PER_TILE)])


@functools.partial(
    pl.kernel,
    mesh=_mesh,
    compiler_params=_sc_params,
    out_type=jax.ShapeDtypeStruct((2 * NP, H), jnp.float32),
    scratch_types=[
        pltpu.VMEM((2, K, C), jnp.int32),
        pltpu.VMEM((2, K, C), jnp.int32),
        pltpu.VMEM((2, K, C, H), jnp.float32),
        pltpu.VMEM_SHARED((NP, H), jnp.float32),
        pltpu.VMEM_SHARED((NP, H), jnp.float32),
        pltpu.SemaphoreType.DMA,
        pltpu.SemaphoreType.DMA,
        pltpu.SemaphoreType.DMA,
        pltpu.SemaphoreType.DMA,
    ],
)
def _sc_aggregate(ei_hbm, g_hbm, zeros_hbm, out_hbm,
                  src_v, dst_v, rows_v, gtab, acc, isem0, isem1, gsem, ssem):
    c = lax.axis_index("c")
    s = lax.axis_index("s")
    wid = s * 2 + c
    r0 = s * ROWS_PER_TILE
    pltpu.sync_copy(zeros_hbm.at[pl.ds(r0, ROWS_PER_TILE)],
                    acc.at[pl.ds(r0, ROWS_PER_TILE)])
    pltpu.sync_copy(g_hbm.at[pl.ds(r0, ROWS_PER_TILE)],
                    gtab.at[pl.ds(r0, ROWS_PER_TILE)])
    plsc.subcore_barrier()
    base = wid * CHUNKS

    def idx_copy(row0, b, sem):
        return (pltpu.async_copy(ei_hbm.at[0, pl.ds(row0, K)], src_v.at[b],
                                 sem),
                pltpu.async_copy(ei_hbm.at[1, pl.ds(row0, K)], dst_v.at[b],
                                 sem))

    def fire_gathers(b):
        return [pltpu.async_copy(gtab.at[src_v.at[b, j]], rows_v.at[b, j],
                                 gsem)
                for j in range(K)]

    def fire_scatters(b):
        return [pltpu.async_copy(rows_v.at[b, j], acc.at[dst_v.at[b, j]],
                                 ssem, add=True)
                for j in range(K)]

    idx_copy(base, 0, isem0)

    def body(i, carry):
        row0 = base + (2 * i) * K
        i1 = idx_copy(row0 + K, 1, isem1)
        pltpu.make_async_copy(ei_hbm.at[0, pl.ds(row0, K)], src_v.at[0],
                              isem0).wait()
        pltpu.make_async_copy(ei_hbm.at[1, pl.ds(row0, K)], dst_v.at[0],
                              isem0).wait()
        ga = fire_gathers(0)
        for d in ga:
            d.wait()
        sa = fire_scatters(0)
        i1[0].wait()
        i1[1].wait()
        gb = fire_gathers(1)
        for d in gb:
            d.wait()
        sb = fire_scatters(1)
        for d in sa:
            d.wait()
        idx_copy(row0 + 2 * K, 0, isem0)
        for d in sb:
            d.wait()
        return carry

    lax.fori_loop(0, NPAIR, body, 0)
    pltpu.make_async_copy(ei_hbm.at[0, pl.ds(base + CHUNKS, K)], src_v.at[0],
                          isem0).wait()
    pltpu.make_async_copy(ei_hbm.at[1, pl.ds(base + CHUNKS, K)], dst_v.at[0],
                          isem0).wait()
    plsc.subcore_barrier()
    pltpu.sync_copy(acc.at[pl.ds(r0, ROWS_PER_TILE)],
                    out_hbm.at[pl.ds(c * NP + r0, ROWS_PER_TILE)])


Nq = N // 8


def _tc_matmul1(xq_ref, w1b_ref, h1p_ref):
    h1p_ref[...] = jnp.dot(xq_ref[...], w1b_ref[...],
                           preferred_element_type=jnp.float32)


def _tc_first(h1p_ref, degp_ref, g1_ref, dinv_ref):
    deg = degp_ref[0:NPq, :] + degp_ref[NPq:2 * NPq, :] + 1.0
    dinv = lax.rsqrt(deg)
    dinv_ref[...] = dinv
    g1_ref[0:Nq, :] = h1p_ref[...] * dinv[0:Nq, :]
    g1_ref[Nq:NPq, :] = jnp.zeros((NPq - Nq, 8 * H), jnp.float32)


def _tc_mid(s1p_ref, g1_ref, dinv_ref, b1_ref, w2b_ref, g2_ref):
    s = s1p_ref[0:NPq, :] + s1p_ref[NPq:2 * NPq, :] + g1_ref[...]
    h = jnp.maximum(s * dinv_ref[...] + b1_ref[...], 0.0)
    h2 = jnp.dot(h, w2b_ref[...], preferred_element_type=jnp.float32)
    g2_ref[...] = h2 * dinv_ref[...]


def _tc_last(s2p_ref, g2_ref, dinv_ref, b2_ref, out_ref):
    s = s2p_ref[0:NPq, :] + s2p_ref[NPq:2 * NPq, :] + g2_ref[...]
    out_ref[...] = s * dinv_ref[...] + b2_ref[...]


_tc_matmul1_call = pl.pallas_call(
    _tc_matmul1,
    out_shape=jax.ShapeDtypeStruct((Nq, 8 * H), jnp.float32),
)

_tc_first_call = pl.pallas_call(
    _tc_first,
    out_shape=(jax.ShapeDtypeStruct((NPq, 8 * H), jnp.float32),
               jax.ShapeDtypeStruct((NPq, 8 * H), jnp.float32)),
)

_tc_mid_call = pl.pallas_call(
    _tc_mid,
    out_shape=jax.ShapeDtypeStruct((NPq, 8 * H), jnp.float32),
)

_tc_last_call = pl.pallas_call(
    _tc_last,
    out_shape=jax.ShapeDtypeStruct((NPq, 8 * H), jnp.float32),
)


def kernel(x, edge_index, W1, b1, W2, b2):
    ei = jnp.pad(edge_index.reshape(2, E // C, C),
                 ((0, 0), (0, ECHUNKS - E // C), (0, 0)),
                 constant_values=N)
    zeros = jnp.zeros((NP, H), jnp.float32)
    ones = jnp.ones((C, H), jnp.float32)
    W2p = jnp.pad(W2, ((0, 0), (0, H - A)))
    eye8 = jnp.eye(8, dtype=jnp.float32)
    w1b = (eye8[:, None, :, None]
           * W1[None, :, None, :]).reshape(8 * D, 8 * H)
    w2b = (eye8[:, None, :, None]
           * W2p[None, :, None, :]).reshape(8 * H, 8 * H)
    b1p = jnp.tile(b1, 8).reshape(1, 8 * H)
    b2p = jnp.tile(jnp.pad(b2, (0, H - A)), 8).reshape(1, 8 * H)

    h1p = _tc_matmul1_call(x.reshape(Nq, 8 * D), w1b)
    degp = _sc_degree(ei, ones, zeros).reshape(2 * NPq, 8 * H)
    g1p, dinvp = _tc_first_call(h1p, degp)
    s1p = _sc_aggregate(ei, g1p.reshape(NP, H),
                        zeros).reshape(2 * NPq, 8 * H)
    g2p = _tc_mid_call(s1p, g1p, dinvp, b1p, w2b)
    s2p = _sc_aggregate(ei, g2p.reshape(NP, H),
                        zeros).reshape(2 * NPq, 8 * H)
    outp = _tc_last_call(s2p, g2p, dinvp, b2p)
    return outp[0:Nq].reshape(Nq, 8, H)[:, :, 0:A].reshape(N, A)
